# baseline (speedup 1.0000x reference)
import functools

import jax
import jax.numpy as jnp
from jax import lax
from jax.experimental import pallas as pl
from jax.experimental.pallas import tpu as pltpu
from jax.experimental.pallas import tpu_sc as plsc

_NW = 32
_CH = 32768


def _sc_body(x1_hbm, x2_hbm, o_hbm, a_v, b_v, sem1, sem2):
    c = lax.axis_index("c")
    s = lax.axis_index("s")
    wid = s * 2 + c
    n = x1_hbm.shape[0]
    per_w = n // _NW
    base = wid * per_w
    n_chunks = per_w // _CH

    def chunk(j, carry):
        off = base + j * _CH
        cp1 = pltpu.async_copy(x1_hbm.at[pl.ds(off, _CH)], a_v, sem1)
        cp2 = pltpu.async_copy(x2_hbm.at[pl.ds(off, _CH)], b_v, sem2)
        cp1.wait()
        cp2.wait()

        def vec(i, carry2):
            p = i * 16
            a_v[pl.ds(p, 16)] = (a_v[pl.ds(p, 16)] + b_v[pl.ds(p, 16)]) * 0.5
            return carry2

        lax.fori_loop(0, _CH // 16, vec, 0)
        pltpu.sync_copy(a_v, o_hbm.at[pl.ds(off, _CH)])
        return carry

    lax.fori_loop(0, n_chunks, chunk, 0)


def kernel(x_1, x_2):
    rows, cols = x_1.shape
    n = rows * cols
    x1f = x_1.reshape(n)
    x2f = x_2.reshape(n)
    mesh = plsc.VectorSubcoreMesh(core_axis_name="c", subcore_axis_name="s")
    k = functools.partial(
        pl.kernel,
        mesh=mesh,
        out_type=jax.ShapeDtypeStruct((n,), jnp.float32),
        scratch_types=[
            pltpu.VMEM((_CH,), jnp.float32),
            pltpu.VMEM((_CH,), jnp.float32),
            pltpu.SemaphoreType.DMA,
            pltpu.SemaphoreType.DMA,
        ],
    )(_sc_body)
    out = k(x1f, x2f)
    return out.reshape(rows, cols)


# speedup vs baseline: 5.2254x; 5.2254x over previous
import functools

import jax
import jax.numpy as jnp
from jax import lax
from jax.experimental import pallas as pl
from jax.experimental.pallas import tpu as pltpu
from jax.experimental.pallas import tpu_sc as plsc

_BLOCK_ROWS = 2048
_R_SC = 4096
_NW = 32
_CH_ROWS = 64


def _avg_kernel(x1_ref, x2_ref, o_ref):
    o_ref[...] = (x1_ref[...] + x2_ref[...]) * 0.5


def _sc_body(x1_hbm, x2_hbm, o_hbm, a_v, b_v, sem1, sem2):
    wid = lax.axis_index("s") * 2 + lax.axis_index("c")
    rows = x1_hbm.shape[0]
    r_tc = rows - _R_SC
    per_w = _R_SC // _NW
    n_chunks = per_w // _CH_ROWS

    def chunk(j, carry):
        o_off = wid * per_w + j * _CH_ROWS
        i_off = r_tc + o_off
        cp1 = pltpu.async_copy(x1_hbm.at[pl.ds(i_off, _CH_ROWS)], a_v, sem1)
        cp2 = pltpu.async_copy(x2_hbm.at[pl.ds(i_off, _CH_ROWS)], b_v, sem2)
        cp1.wait()
        cp2.wait()

        def row(r, c2):
            def vec(i, c3):
                p = i * 16
                a_v[r, pl.ds(p, 16)] = (
                    a_v[r, pl.ds(p, 16)] + b_v[r, pl.ds(p, 16)]
                ) * 0.5
                return c3

            lax.fori_loop(0, a_v.shape[1] // 16, vec, 0)
            return c2

        lax.fori_loop(0, _CH_ROWS, row, 0)
        pltpu.sync_copy(a_v, o_hbm.at[pl.ds(o_off, _CH_ROWS)])
        return carry

    lax.fori_loop(0, n_chunks, chunk, 0)


def kernel(x_1, x_2):
    rows, cols = x_1.shape
    r_tc = rows - _R_SC

    mesh = plsc.VectorSubcoreMesh(core_axis_name="c", subcore_axis_name="s")
    sc_call = functools.partial(
        pl.kernel,
        mesh=mesh,
        out_type=jax.ShapeDtypeStruct((_R_SC, cols), jnp.float32),
        scratch_types=[
            pltpu.VMEM((_CH_ROWS, cols), jnp.float32),
            pltpu.VMEM((_CH_ROWS, cols), jnp.float32),
            pltpu.SemaphoreType.DMA,
            pltpu.SemaphoreType.DMA,
        ],
    )(_sc_body)
    sc_out = sc_call(x_1, x_2)

    spec = pl.BlockSpec((_BLOCK_ROWS, cols), lambda i: (i, 0))
    tc_out = pl.pallas_call(
        _avg_kernel,
        grid=(r_tc // _BLOCK_ROWS,),
        in_specs=[spec, spec],
        out_specs=spec,
        out_shape=jax.ShapeDtypeStruct((rows, cols), x_1.dtype),
        compiler_params=pltpu.CompilerParams(
            dimension_semantics=("arbitrary",),
        ),
    )(x_1, x_2)

    return lax.dynamic_update_slice(tc_out, sc_out, (r_tc, 0))


# speedup vs baseline: 7.2071x; 1.3792x over previous
import jax
import jax.numpy as jnp
from jax.experimental import pallas as pl
from jax.experimental.pallas import tpu as pltpu

_BLOCK_ROWS = 2048


def _avg_kernel(x1_ref, x2_ref, o_ref):
    o_ref[...] = (x1_ref[...] + x2_ref[...]) * 0.5


def kernel(x_1, x_2):
    rows, cols = x_1.shape
    grid = (rows // _BLOCK_ROWS,)
    spec = pl.BlockSpec((_BLOCK_ROWS, cols), lambda i: (i, 0))
    return pl.pallas_call(
        _avg_kernel,
        grid=grid,
        in_specs=[spec, spec],
        out_specs=spec,
        out_shape=jax.ShapeDtypeStruct((rows, cols), x_1.dtype),
        compiler_params=pltpu.CompilerParams(
            dimension_semantics=("parallel",),
        ),
    )(x_1, x_2)


# speedup vs baseline: 7.2341x; 1.0038x over previous
import jax
import jax.numpy as jnp
from jax.experimental import pallas as pl
from jax.experimental.pallas import tpu as pltpu

_BLOCK_ROWS = 2048


def _avg_kernel(x1_ref, x2_ref, o_ref):
    o_ref[...] = (x1_ref[...] + x2_ref[...]) * 0.5


def kernel(x_1, x_2):
    rows, cols = x_1.shape
    grid = (rows // _BLOCK_ROWS,)
    spec = pl.BlockSpec((_BLOCK_ROWS, cols), lambda i: (i, 0))
    return pl.pallas_call(
        _avg_kernel,
        grid=grid,
        in_specs=[spec, spec],
        out_specs=spec,
        out_shape=jax.ShapeDtypeStruct((rows, cols), x_1.dtype),
        compiler_params=pltpu.CompilerParams(
            dimension_semantics=("arbitrary",),
        ),
    )(x_1, x_2)
